# Initial kernel scaffold; baseline (speedup 1.0000x reference)
import jax
import jax.numpy as jnp
from jax.experimental import pallas as pl


def kernel(inputs, targets, W, b):
    raise NotImplementedError("write your pallas kernel here")


# speedup vs baseline: 10.4514x; 10.4514x over previous
import jax
import jax.numpy as jnp
from jax.experimental import pallas as pl
from jax.experimental.pallas import tpu as pltpu

_B = 512
_D_IN = 1024
_D_OUT = 128


def _fused_body(x_ref, t_ref, w_ref, b_ref, out_ref):
    h = jnp.dot(x_ref[...], w_ref[...], preferred_element_type=jnp.float32)
    h = h + b_ref[...]
    sq = jnp.sum(h * h, axis=1)
    g = jax.lax.dot_general(
        h, h, (((1,), (1,)), ((), ())), preferred_element_type=jnp.float32
    )
    d2 = jnp.maximum(sq[:, None] + sq[None, :] - 2.0 * g, 0.0)
    dist = jnp.sqrt(d2)

    t = t_ref[...]
    same = jnp.transpose(t) == t
    ri = jax.lax.broadcasted_iota(jnp.int32, (_B, _B), 0)
    ci = jax.lax.broadcasted_iota(jnp.int32, (_B, _B), 1)
    not_self = ri != ci
    pos_mask = same & not_self
    neg_mask = jnp.logical_not(same)

    hp = jnp.max(jnp.where(pos_mask, dist, -1e30), axis=1)
    hn = jnp.min(jnp.where(neg_mask, dist, 1e30), axis=1)
    diff = hp - hn
    sp = jnp.log1p(jnp.exp(-jnp.abs(diff))) + jnp.maximum(diff, 0.0)
    out_ref[...] = jnp.sum(sp, keepdims=True).reshape(1, 1)


def kernel(inputs, targets, W, b):
    t2 = targets.astype(jnp.int32).reshape(1, _B)
    b2 = b.reshape(1, _D_OUT)
    out = pl.pallas_call(
        _fused_body,
        out_shape=jax.ShapeDtypeStruct((1, 1), jnp.float32),
    )(inputs, t2, W, b2)
    return out[0, 0]
